# baseline (speedup 1.0000x reference)
import functools

import jax
import jax.numpy as jnp
from jax import lax
from jax.experimental import pallas as pl
from jax.experimental.pallas import tpu as pltpu
from jax.experimental.pallas import tpu_sc as plsc

_N = 10000
_NPAD = 10240
_E = 320000
_D = 128
_G = 64
_NC = 2
_NS = 16
_NW = _NC * _NS
_EPW = _E // _NW
_CH = 128
_NCHUNK = 80
_EROWS = _NW * _NCHUNK
_EPAD = _EROWS * _CH
_RPT = _NPAD // _NS


def _sc_mesh():
    return plsc.VectorSubcoreMesh(core_axis_name="c", subcore_axis_name="s")


def _deg_kernel(dstp, zeros16):
    @functools.partial(
        pl.kernel,
        mesh=_sc_mesh(),
        out_type=jax.ShapeDtypeStruct((_NC, _NPAD, 16), jnp.float32),
        compiler_params=pltpu.CompilerParams(use_tc_tiling_on_sc=False),
        scratch_types=[
            pltpu.VMEM((_CH,), jnp.int32),
            pltpu.VMEM((_CH,), jnp.int32),
            pltpu.VMEM((_CH, 16), jnp.float32),
            pltpu.VMEM_SHARED((_NPAD, 16), jnp.float32),
            pltpu.SemaphoreType.DMA,
            pltpu.SemaphoreType.DMA,
            pltpu.SemaphoreType.DMA,
            pltpu.SemaphoreType.DMA,
        ],
    )
    def k(dst_hbm, z_hbm, out_hbm, didx0, didx1, ones_v, acc,
          is0, is1, ss0, ss1):
        cid = lax.axis_index("c")
        sid = lax.axis_index("s")
        wid = sid * _NC + cid
        base = wid * _NCHUNK * _CH
        one_row = jnp.where(
            lax.iota(jnp.int32, 16) == 0, jnp.float32(1.0), jnp.float32(0.0)
        )

        def fill(i, carry):
            ones_v[i, :] = one_row
            return carry

        lax.fori_loop(0, _CH, fill, 0)
        pltpu.sync_copy(z_hbm, acc.at[pl.ds(sid * _RPT, _RPT)])
        plsc.subcore_barrier()

        didx = (didx0, didx1)
        isem = (is0, is1)
        ssem = (ss0, ss1)

        def idx_load(cc, p):
            pltpu.async_copy(
                dst_hbm.at[pl.ds(base + cc * _CH, _CH)], didx[p], isem[p]
            )

        def wait_idx(p):
            pltpu.make_async_copy(
                dst_hbm.at[pl.ds(base, _CH)], didx[p], isem[p]
            ).wait()

        def fire_scat(p):
            pltpu.async_copy(ones_v, acc.at[didx[p]], ssem[p], add=True)

        def wait_scat(p):
            pltpu.make_async_copy(ones_v, acc.at[didx[p]], ssem[p]).wait()

        idx_load(0, 0)
        idx_load(1, 1)
        wait_idx(0)
        fire_scat(0)

        @pl.loop(1, _NCHUNK - 1, step=2)
        def _(c):
            for dc, p in ((0, 1), (1, 0)):
                cc = c + dc
                wait_idx(p)
                fire_scat(p)
                wait_scat(1 - p)
                idx_load(cc + 1, 1 - p)

        wait_idx(1)
        fire_scat(1)
        wait_scat(0)
        wait_scat(1)
        plsc.subcore_barrier()
        pltpu.sync_copy(
            acc.at[pl.ds(sid * _RPT, _RPT)],
            out_hbm.at[cid, pl.ds(sid * _RPT, _RPT)],
        )

    return k(dstp, zeros16)


_EPWP = _NCHUNK * _CH


def _edge_scatter(g, srcp, dstp):
    @functools.partial(
        pl.kernel,
        mesh=_sc_mesh(),
        out_type=jax.ShapeDtypeStruct((_NC, _NPAD, _D), jnp.float32),
        scratch_types=[
            pltpu.VMEM((_EPWP,), jnp.int32),
            pltpu.VMEM((_CH,), jnp.int32),
            pltpu.VMEM((_CH,), jnp.int32),
            pltpu.VMEM((_CH, _D), jnp.float32),
            pltpu.VMEM((_CH, _D), jnp.float32),
            pltpu.VMEM_SHARED((_NPAD, _D), jnp.float32),
            pltpu.SemaphoreType.DMA,
            pltpu.SemaphoreType.DMA,
            pltpu.SemaphoreType.DMA,
            pltpu.SemaphoreType.DMA,
        ],
    )
    def k(g_hbm, src_hbm, dst_hbm, out_hbm,
          sidx, didx0, didx1, rows0, rows1, acc, gs0, gs1, is0, is1):
        cid = lax.axis_index("c")
        sid = lax.axis_index("s")
        wid = sid * _NC + cid
        base = wid * _EPWP
        zrow = jnp.zeros((16,), jnp.float32)

        def zfill(i, carry):
            for j in range(8):
                rows0[i, pl.ds(j * 16, 16)] = zrow
            return carry

        lax.fori_loop(0, _CH, zfill, 0)
        for r in range(_RPT // _CH):
            pltpu.sync_copy(
                rows0, acc.at[pl.ds(sid * _RPT + r * _CH, _CH)]
            )
        pltpu.sync_copy(src_hbm.at[pl.ds(base, _EPWP)], sidx)
        plsc.subcore_barrier()

        rows = (rows0, rows1)
        gsem = (gs0, gs1)
        didx = (didx0, didx1)
        isem = (is0, is1)

        def fire_gather(cc, p):
            pltpu.async_copy(
                g_hbm.at[sidx.at[pl.ds(cc * _CH, _CH)]], rows[p], gsem[p]
            )

        def wait_gather(p):
            pltpu.make_async_copy(
                g_hbm.at[sidx.at[pl.ds(0, _CH)]], rows[p], gsem[p]
            ).wait()

        def idx_load(cc, p):
            pltpu.async_copy(
                dst_hbm.at[pl.ds(base + cc * _CH, _CH)], didx[p], isem[p]
            )

        def wait_idx(p):
            pltpu.make_async_copy(
                dst_hbm.at[pl.ds(base, _CH)], didx[p], isem[p]
            ).wait()

        idx_load(0, 0)
        idx_load(1, 1)
        fire_gather(0, 0)

        def pair(i, carry):
            c = i * 2
            for dc, p in ((0, 0), (1, 1)):
                cc = c + dc
                wait_gather(p)
                fire_gather(cc + 1, 1 - p)
                wait_idx(p)
                pltpu.sync_copy(rows[p], acc.at[didx[p]], add=True)
                idx_load(cc + 2, p)
            return carry

        lax.fori_loop(0, (_NCHUNK - 2) // 2, pair, 0)

        wait_gather(0)
        fire_gather(_NCHUNK - 1, 1)
        wait_idx(0)
        pltpu.sync_copy(rows[0], acc.at[didx[0]], add=True)
        wait_gather(1)
        wait_idx(1)
        pltpu.sync_copy(rows[1], acc.at[didx[1]], add=True)
        plsc.subcore_barrier()
        pltpu.sync_copy(
            acc.at[pl.ds(sid * _RPT, _RPT)],
            out_hbm.at[cid, pl.ds(sid * _RPT, _RPT)],
        )

    return k(g, srcp, dstp)


_BLK = 2048
_GRID = _NPAD // _BLK


def _dinv_of(deg_ref):
    deg = jnp.sum(deg_ref[...], axis=(0, 2)) + 1.0
    return lax.rsqrt(deg)


def _first_body(deg_ref, x_ref, w_ref, o_ref):
    dinv = _dinv_of(deg_ref)
    h = jnp.dot(x_ref[...], w_ref[...], preferred_element_type=jnp.float32)
    o_ref[...] = h * dinv[:, None]


def _mid_body(deg_ref, acc_ref, g_ref, w_ref, b_ref, o_ref):
    dinv = _dinv_of(deg_ref)
    s = acc_ref[0] + acc_ref[1] + g_ref[...]
    h = jnp.maximum(s * dinv[:, None] + b_ref[...], 0.0)
    o_ref[...] = (
        jnp.dot(h, w_ref[...], preferred_element_type=jnp.float32) * dinv[:, None]
    )


def _pool_body(deg_ref, acc_ref, g_ref, b_ref, batch_ref, lw_ref, lb_ref,
               o_ref, sums_ref, cnt_ref):
    i = pl.program_id(0)
    dinv = _dinv_of(deg_ref)
    s = acc_ref[0] + acc_ref[1] + g_ref[...]
    h = jnp.maximum(s * dinv[:, None] + b_ref[...], 0.0)
    b_blk = batch_ref[0, 0, :]
    row_id = lax.broadcasted_iota(jnp.int32, (_BLK, _G), 0) + i * _BLK
    onehot = (
        (b_blk[:, None] == lax.broadcasted_iota(jnp.int32, (_BLK, _G), 1))
        & (row_id < _N)
    ).astype(jnp.float32)
    psum = lax.dot_general(
        onehot, h, (((0,), (0,)), ((), ())), preferred_element_type=jnp.float32
    )
    pcnt = jnp.sum(onehot, axis=0)

    @pl.when(i == 0)
    def _():
        sums_ref[...] = psum
        cnt_ref[...] = pcnt

    @pl.when(i > 0)
    def _():
        sums_ref[...] += psum
        cnt_ref[...] += pcnt

    @pl.when(i == _GRID - 1)
    def _():
        mean = sums_ref[...] / jnp.maximum(cnt_ref[...], 1.0)[:, None]
        o_ref[...] = (
            jnp.dot(mean, lw_ref[...], preferred_element_type=jnp.float32)
            + lb_ref[...]
        )


def _deg_spec():
    return pl.BlockSpec((_NC, _BLK, 16), lambda i: (0, i, 0))


def _row_spec():
    return pl.BlockSpec((_BLK, _D), lambda i: (i, 0))


def _acc_spec():
    return pl.BlockSpec((_NC, _BLK, _D), lambda i: (0, i, 0))


def _full_spec(shape):
    nd = len(shape)
    return pl.BlockSpec(shape, lambda i: (0,) * nd)


def _tc_first(degp, x, W):
    return pl.pallas_call(
        _first_body,
        grid=(_GRID,),
        in_specs=[_deg_spec(), _row_spec(), _full_spec((_D, _D))],
        out_specs=_row_spec(),
        out_shape=jax.ShapeDtypeStruct((_NPAD, _D), jnp.float32),
    )(degp, x, W)


def _tc_mid(degp, accp, g, W, b):
    return pl.pallas_call(
        _mid_body,
        grid=(_GRID,),
        in_specs=[_deg_spec(), _acc_spec(), _row_spec(),
                  _full_spec((_D, _D)), _full_spec((1, _D))],
        out_specs=_row_spec(),
        out_shape=jax.ShapeDtypeStruct((_NPAD, _D), jnp.float32),
    )(degp, accp, g, W, b)


def _tc_pool(degp, accp, g, b, batch3d, lw, lb):
    return pl.pallas_call(
        _pool_body,
        grid=(_GRID,),
        in_specs=[_deg_spec(), _acc_spec(), _row_spec(), _full_spec((1, _D)),
                  pl.BlockSpec((1, 1, _BLK), lambda i: (i, 0, 0)),
                  _full_spec((_D, _D)), _full_spec((1, _D))],
        out_specs=_full_spec((_G, _D)),
        out_shape=jax.ShapeDtypeStruct((_G, _D), jnp.float32),
        scratch_shapes=[
            pltpu.VMEM((_G, _D), jnp.float32),
            pltpu.VMEM((_G,), jnp.float32),
        ],
    )(degp, accp, g, b, batch3d, lw, lb)


def kernel(x, edge_index, batch, W1, b1, W2, b2, lin_W, lin_b):
    pad = jnp.full((_EPAD - _E,), _NPAD - 1, jnp.int32)
    src = edge_index[0].astype(jnp.int32)
    dst = edge_index[1].astype(jnp.int32)
    srcp = jnp.concatenate([src, pad])
    dstp = jnp.concatenate([dst, pad])
    x = jnp.pad(x, ((0, _NPAD - _N), (0, 0)))
    batch3d = (
        jnp.pad(batch.astype(jnp.int32), (0, _NPAD - _N))
        .reshape(_GRID, 1, _BLK)
    )
    zeros16 = jnp.zeros((_RPT, 16), jnp.float32)
    b1r = b1.reshape(1, _D)
    b2r = b2.reshape(1, _D)
    lbr = lin_b.reshape(1, _D)

    degp = _deg_kernel(dstp, zeros16)
    g1 = _tc_first(degp, x, W1)
    acc1 = _edge_scatter(g1, srcp, dstp)
    g2 = _tc_mid(degp, acc1, g1, W2, b1r)
    acc2 = _edge_scatter(g2, srcp, dstp)
    return _tc_pool(degp, acc2, g2, b2r, batch3d, lin_W, lbr)


# speedup vs baseline: 1.0000x; 1.0000x over previous
import functools

import jax
import jax.numpy as jnp
from jax import lax
from jax.experimental import pallas as pl
from jax.experimental.pallas import tpu as pltpu
from jax.experimental.pallas import tpu_sc as plsc

_N = 10000
_NPAD = 10240
_E = 320000
_D = 128
_G = 64
_NC = 2
_NS = 16
_NW = _NC * _NS
_EPW = _E // _NW
_CH = 128
_NCHUNK = 80
_EROWS = _NW * _NCHUNK
_EPAD = _EROWS * _CH
_RPT = _NPAD // _NS


def _sc_mesh():
    return plsc.VectorSubcoreMesh(core_axis_name="c", subcore_axis_name="s")


def _deg_kernel(dstp, zeros16):
    @functools.partial(
        pl.kernel,
        mesh=_sc_mesh(),
        out_type=jax.ShapeDtypeStruct((_NC, _NPAD, 16), jnp.float32),
        compiler_params=pltpu.CompilerParams(use_tc_tiling_on_sc=False),
        scratch_types=[
            pltpu.VMEM((_CH,), jnp.int32),
            pltpu.VMEM((_CH,), jnp.int32),
            pltpu.VMEM((_CH, 16), jnp.float32),
            pltpu.VMEM_SHARED((_NPAD, 16), jnp.float32),
            pltpu.SemaphoreType.DMA,
            pltpu.SemaphoreType.DMA,
            pltpu.SemaphoreType.DMA,
            pltpu.SemaphoreType.DMA,
        ],
    )
    def k(dst_hbm, z_hbm, out_hbm, didx0, didx1, ones_v, acc,
          is0, is1, ss0, ss1):
        cid = lax.axis_index("c")
        sid = lax.axis_index("s")
        wid = sid * _NC + cid
        base = wid * _NCHUNK * _CH
        one_row = jnp.where(
            lax.iota(jnp.int32, 16) == 0, jnp.float32(1.0), jnp.float32(0.0)
        )

        def fill(i, carry):
            ones_v[i, :] = one_row
            return carry

        lax.fori_loop(0, _CH, fill, 0)
        pltpu.sync_copy(z_hbm, acc.at[pl.ds(sid * _RPT, _RPT)])
        plsc.subcore_barrier()

        didx = (didx0, didx1)
        isem = (is0, is1)
        ssem = (ss0, ss1)

        def idx_load(cc, p):
            pltpu.async_copy(
                dst_hbm.at[pl.ds(base + cc * _CH, _CH)], didx[p], isem[p]
            )

        def wait_idx(p):
            pltpu.make_async_copy(
                dst_hbm.at[pl.ds(base, _CH)], didx[p], isem[p]
            ).wait()

        def fire_scat(p):
            pltpu.async_copy(ones_v, acc.at[didx[p]], ssem[p], add=True)

        def wait_scat(p):
            pltpu.make_async_copy(ones_v, acc.at[didx[p]], ssem[p]).wait()

        idx_load(0, 0)
        idx_load(1, 1)
        wait_idx(0)
        fire_scat(0)

        @pl.loop(1, _NCHUNK - 1, step=2)
        def _(c):
            for dc, p in ((0, 1), (1, 0)):
                cc = c + dc
                wait_idx(p)
                fire_scat(p)
                wait_scat(1 - p)
                idx_load(cc + 1, 1 - p)

        wait_idx(1)
        fire_scat(1)
        wait_scat(0)
        wait_scat(1)
        plsc.subcore_barrier()
        pltpu.sync_copy(
            acc.at[pl.ds(sid * _RPT, _RPT)],
            out_hbm.at[cid, pl.ds(sid * _RPT, _RPT)],
        )

    return k(dstp, zeros16)


_EPWP = _NCHUNK * _CH


def _edge_scatter(g, srcp, dstp):
    @functools.partial(
        pl.kernel,
        mesh=_sc_mesh(),
        out_type=jax.ShapeDtypeStruct((_NC, _NPAD, _D), jnp.float32),
        scratch_types=[
            pltpu.VMEM((_CH,), jnp.int32),
            pltpu.VMEM((_CH,), jnp.int32),
            pltpu.VMEM((_CH,), jnp.int32),
            pltpu.VMEM((_CH,), jnp.int32),
            pltpu.VMEM((_CH, _D), jnp.float32),
            pltpu.VMEM((_CH, _D), jnp.float32),
            pltpu.VMEM_SHARED((_NPAD, _D), jnp.float32),
            pltpu.SemaphoreType.DMA,
            pltpu.SemaphoreType.DMA,
            pltpu.SemaphoreType.DMA,
            pltpu.SemaphoreType.DMA,
        ],
    )
    def k(g_hbm, src_hbm, dst_hbm, out_hbm,
          sidx0, sidx1, didx0, didx1, rows0, rows1, acc, gs0, gs1, is0, is1):
        cid = lax.axis_index("c")
        sid = lax.axis_index("s")
        wid = sid * _NC + cid
        base = wid * _EPWP
        zrow = jnp.zeros((16,), jnp.float32)

        def zfill(i, carry):
            for j in range(8):
                rows0[i, pl.ds(j * 16, 16)] = zrow
            return carry

        lax.fori_loop(0, _CH, zfill, 0)
        for r in range(_RPT // _CH):
            pltpu.sync_copy(
                rows0, acc.at[pl.ds(sid * _RPT + r * _CH, _CH)]
            )
        plsc.subcore_barrier()

        rows = (rows0, rows1)
        gsem = (gs0, gs1)
        sidx = (sidx0, sidx1)
        didx = (didx0, didx1)
        isem = (is0, is1)

        def idx_load(cc, p):
            off = base + cc * _CH
            pltpu.async_copy(src_hbm.at[pl.ds(off, _CH)], sidx[p], isem[p])
            pltpu.async_copy(dst_hbm.at[pl.ds(off, _CH)], didx[p], isem[p])

        def wait_idx(p):
            pltpu.make_async_copy(
                src_hbm.at[pl.ds(base, _CH)], sidx[p], isem[p]
            ).wait()
            pltpu.make_async_copy(
                dst_hbm.at[pl.ds(base, _CH)], didx[p], isem[p]
            ).wait()

        def fire_gather(p):
            pltpu.async_copy(g_hbm.at[sidx[p]], rows[p], gsem[p])

        def wait_gather(p):
            pltpu.make_async_copy(g_hbm.at[sidx[p]], rows[p], gsem[p]).wait()

        idx_load(0, 0)
        idx_load(1, 1)
        wait_idx(0)
        fire_gather(0)

        def pair(i, carry):
            c = i * 2
            for dc, p in ((0, 0), (1, 1)):
                cc = c + dc
                wait_gather(p)
                wait_idx(1 - p)
                fire_gather(1 - p)
                pltpu.sync_copy(rows[p], acc.at[didx[p]], add=True)
                idx_load(cc + 2, p)
            return carry

        lax.fori_loop(0, (_NCHUNK - 2) // 2, pair, 0)

        wait_gather(0)
        wait_idx(1)
        fire_gather(1)
        pltpu.sync_copy(rows[0], acc.at[didx[0]], add=True)
        wait_gather(1)
        pltpu.sync_copy(rows[1], acc.at[didx[1]], add=True)
        plsc.subcore_barrier()
        pltpu.sync_copy(
            acc.at[pl.ds(sid * _RPT, _RPT)],
            out_hbm.at[cid, pl.ds(sid * _RPT, _RPT)],
        )

    return k(g, srcp, dstp)


_BLK = 2048
_GRID = _NPAD // _BLK


def _dinv_of(deg_ref):
    deg = jnp.sum(deg_ref[...], axis=(0, 2)) + 1.0
    return lax.rsqrt(deg)


def _first_body(deg_ref, x_ref, w_ref, o_ref):
    dinv = _dinv_of(deg_ref)
    h = jnp.dot(x_ref[...], w_ref[...], preferred_element_type=jnp.float32)
    o_ref[...] = h * dinv[:, None]


def _mid_body(deg_ref, acc_ref, g_ref, w_ref, b_ref, o_ref):
    dinv = _dinv_of(deg_ref)
    s = acc_ref[0] + acc_ref[1] + g_ref[...]
    h = jnp.maximum(s * dinv[:, None] + b_ref[...], 0.0)
    o_ref[...] = (
        jnp.dot(h, w_ref[...], preferred_element_type=jnp.float32) * dinv[:, None]
    )


def _pool_body(deg_ref, acc_ref, g_ref, b_ref, batch_ref, lw_ref, lb_ref,
               o_ref, sums_ref, cnt_ref):
    i = pl.program_id(0)
    dinv = _dinv_of(deg_ref)
    s = acc_ref[0] + acc_ref[1] + g_ref[...]
    h = jnp.maximum(s * dinv[:, None] + b_ref[...], 0.0)
    b_blk = batch_ref[0, 0, :]
    row_id = lax.broadcasted_iota(jnp.int32, (_BLK, _G), 0) + i * _BLK
    onehot = (
        (b_blk[:, None] == lax.broadcasted_iota(jnp.int32, (_BLK, _G), 1))
        & (row_id < _N)
    ).astype(jnp.float32)
    psum = lax.dot_general(
        onehot, h, (((0,), (0,)), ((), ())), preferred_element_type=jnp.float32
    )
    pcnt = jnp.sum(onehot, axis=0)

    @pl.when(i == 0)
    def _():
        sums_ref[...] = psum
        cnt_ref[...] = pcnt

    @pl.when(i > 0)
    def _():
        sums_ref[...] += psum
        cnt_ref[...] += pcnt

    @pl.when(i == _GRID - 1)
    def _():
        mean = sums_ref[...] / jnp.maximum(cnt_ref[...], 1.0)[:, None]
        o_ref[...] = (
            jnp.dot(mean, lw_ref[...], preferred_element_type=jnp.float32)
            + lb_ref[...]
        )


def _deg_spec():
    return pl.BlockSpec((_NC, _BLK, 16), lambda i: (0, i, 0))


def _row_spec():
    return pl.BlockSpec((_BLK, _D), lambda i: (i, 0))


def _acc_spec():
    return pl.BlockSpec((_NC, _BLK, _D), lambda i: (0, i, 0))


def _full_spec(shape):
    nd = len(shape)
    return pl.BlockSpec(shape, lambda i: (0,) * nd)


def _tc_first(degp, x, W):
    return pl.pallas_call(
        _first_body,
        grid=(_GRID,),
        in_specs=[_deg_spec(), _row_spec(), _full_spec((_D, _D))],
        out_specs=_row_spec(),
        out_shape=jax.ShapeDtypeStruct((_NPAD, _D), jnp.float32),
    )(degp, x, W)


def _tc_mid(degp, accp, g, W, b):
    return pl.pallas_call(
        _mid_body,
        grid=(_GRID,),
        in_specs=[_deg_spec(), _acc_spec(), _row_spec(),
                  _full_spec((_D, _D)), _full_spec((1, _D))],
        out_specs=_row_spec(),
        out_shape=jax.ShapeDtypeStruct((_NPAD, _D), jnp.float32),
    )(degp, accp, g, W, b)


def _tc_pool(degp, accp, g, b, batch3d, lw, lb):
    return pl.pallas_call(
        _pool_body,
        grid=(_GRID,),
        in_specs=[_deg_spec(), _acc_spec(), _row_spec(), _full_spec((1, _D)),
                  pl.BlockSpec((1, 1, _BLK), lambda i: (i, 0, 0)),
                  _full_spec((_D, _D)), _full_spec((1, _D))],
        out_specs=_full_spec((_G, _D)),
        out_shape=jax.ShapeDtypeStruct((_G, _D), jnp.float32),
        scratch_shapes=[
            pltpu.VMEM((_G, _D), jnp.float32),
            pltpu.VMEM((_G,), jnp.float32),
        ],
    )(degp, accp, g, b, batch3d, lw, lb)


def kernel(x, edge_index, batch, W1, b1, W2, b2, lin_W, lin_b):
    pad = jnp.full((_EPAD - _E,), _NPAD - 1, jnp.int32)
    src = edge_index[0].astype(jnp.int32)
    dst = edge_index[1].astype(jnp.int32)
    srcp = jnp.concatenate([src, pad])
    dstp = jnp.concatenate([dst, pad])
    x = jnp.pad(x, ((0, _NPAD - _N), (0, 0)))
    batch3d = (
        jnp.pad(batch.astype(jnp.int32), (0, _NPAD - _N))
        .reshape(_GRID, 1, _BLK)
    )
    zeros16 = jnp.zeros((_RPT, 16), jnp.float32)
    b1r = b1.reshape(1, _D)
    b2r = b2.reshape(1, _D)
    lbr = lin_b.reshape(1, _D)

    degp = _deg_kernel(dstp, zeros16)
    g1 = _tc_first(degp, x, W1)
    acc1 = _edge_scatter(g1, srcp, dstp)
    g2 = _tc_mid(degp, acc1, g1, W2, b1r)
    acc2 = _edge_scatter(g2, srcp, dstp)
    return _tc_pool(degp, acc2, g2, b2r, batch3d, lin_W, lbr)


# speedup vs baseline: 1.9219x; 1.9219x over previous
import functools

import jax
import jax.numpy as jnp
from jax import lax
from jax.experimental import pallas as pl
from jax.experimental.pallas import tpu as pltpu
from jax.experimental.pallas import tpu_sc as plsc

_N = 10000
_NPAD = 10240
_E = 320000
_D = 128
_G = 64
_NC = 2
_NS = 16
_NW = _NC * _NS
_EPW = _E // _NW
_CH = 128
_NFULL = _EPW // _CH
_REM = _EPW - _NFULL * _CH
_RPT = _NPAD // _NS


def _sc_mesh():
    return plsc.VectorSubcoreMesh(core_axis_name="c", subcore_axis_name="s")


def _deg_kernel(dst, zeros16):
    @functools.partial(
        pl.kernel,
        mesh=_sc_mesh(),
        out_type=jax.ShapeDtypeStruct((_NC, _NPAD, 16), jnp.float32),
        compiler_params=pltpu.CompilerParams(use_tc_tiling_on_sc=False),
        scratch_types=[
            pltpu.VMEM((_CH,), jnp.int32),
            pltpu.VMEM((_REM,), jnp.int32),
            pltpu.VMEM((_CH, 16), jnp.float32),
            pltpu.VMEM_SHARED((_NPAD, 16), jnp.float32),
        ],
    )
    def k(dst_hbm, z_hbm, out_hbm, didx, didx2, ones_v, acc):
        cid = lax.axis_index("c")
        sid = lax.axis_index("s")
        wid = sid * _NC + cid
        base = wid * _EPW
        one_row = jnp.where(
            lax.iota(jnp.int32, 16) == 0, jnp.float32(1.0), jnp.float32(0.0)
        )

        def fill(i, carry):
            ones_v[i, :] = one_row
            return carry

        lax.fori_loop(0, _CH, fill, 0)
        pltpu.sync_copy(z_hbm, acc.at[pl.ds(sid * _RPT, _RPT)])
        plsc.subcore_barrier()

        def body(c, carry):
            off = base + c * _CH
            pltpu.sync_copy(dst_hbm.at[pl.ds(off, _CH)], didx)
            pltpu.sync_copy(ones_v, acc.at[didx], add=True)
            return carry

        lax.fori_loop(0, _NFULL, body, 0)
        off = base + _NFULL * _CH
        pltpu.sync_copy(dst_hbm.at[pl.ds(off, _REM)], didx2)
        pltpu.sync_copy(ones_v.at[pl.ds(0, _REM)], acc.at[didx2], add=True)
        plsc.subcore_barrier()
        pltpu.sync_copy(
            acc.at[pl.ds(sid * _RPT, _RPT)],
            out_hbm.at[cid, pl.ds(sid * _RPT, _RPT)],
        )

    return k(dst, zeros16)


def _edge_scatter(g, src, dst, zeros_rows):
    @functools.partial(
        pl.kernel,
        mesh=_sc_mesh(),
        out_type=jax.ShapeDtypeStruct((_NC, _NPAD, _D), jnp.float32),
        scratch_types=[
            pltpu.VMEM((_CH,), jnp.int32),
            pltpu.VMEM((_CH,), jnp.int32),
            pltpu.VMEM((_REM,), jnp.int32),
            pltpu.VMEM((_REM,), jnp.int32),
            pltpu.VMEM((_CH, _D), jnp.float32),
            pltpu.VMEM_SHARED((_NPAD, _D), jnp.float32),
            pltpu.SemaphoreType.DMA,
        ],
    )
    def k(g_hbm, src_hbm, dst_hbm, z_hbm, out_hbm,
          sidx, didx, sidx2, didx2, rows, acc, sem):
        cid = lax.axis_index("c")
        sid = lax.axis_index("s")
        wid = sid * _NC + cid
        base = wid * _EPW
        pltpu.sync_copy(z_hbm, acc.at[pl.ds(sid * _RPT, _RPT)])
        plsc.subcore_barrier()

        def body(c, carry):
            off = base + c * _CH
            pltpu.sync_copy(src_hbm.at[pl.ds(off, _CH)], sidx)
            pltpu.sync_copy(dst_hbm.at[pl.ds(off, _CH)], didx)
            pltpu.async_copy(g_hbm.at[sidx], rows, sem).wait()
            pltpu.sync_copy(rows, acc.at[didx], add=True)
            return carry

        lax.fori_loop(0, _NFULL, body, 0)
        off = base + _NFULL * _CH
        pltpu.sync_copy(src_hbm.at[pl.ds(off, _REM)], sidx2)
        pltpu.sync_copy(dst_hbm.at[pl.ds(off, _REM)], didx2)
        pltpu.async_copy(g_hbm.at[sidx2], rows.at[pl.ds(0, _REM)], sem).wait()
        pltpu.sync_copy(rows.at[pl.ds(0, _REM)], acc.at[didx2], add=True)
        plsc.subcore_barrier()
        pltpu.sync_copy(
            acc.at[pl.ds(sid * _RPT, _RPT)],
            out_hbm.at[cid, pl.ds(sid * _RPT, _RPT)],
        )

    return k(g, src, dst, zeros_rows)


_BLK = 2048
_GRID = _NPAD // _BLK


def _dinv_of(deg_ref):
    deg = jnp.sum(deg_ref[...], axis=(0, 2)) + 1.0
    return lax.rsqrt(deg)


def _first_body(deg_ref, x_ref, w_ref, o_ref):
    dinv = _dinv_of(deg_ref)
    h = jnp.dot(x_ref[...], w_ref[...], preferred_element_type=jnp.float32)
    o_ref[...] = h * dinv[:, None]


def _mid_body(deg_ref, acc_ref, g_ref, w_ref, b_ref, o_ref):
    dinv = _dinv_of(deg_ref)
    s = acc_ref[0] + acc_ref[1] + g_ref[...]
    h = jnp.maximum(s * dinv[:, None] + b_ref[...], 0.0)
    o_ref[...] = (
        jnp.dot(h, w_ref[...], preferred_element_type=jnp.float32) * dinv[:, None]
    )


def _pool_body(deg_ref, acc_ref, g_ref, b_ref, batch_ref, lw_ref, lb_ref,
               o_ref, sums_ref, cnt_ref):
    i = pl.program_id(0)
    dinv = _dinv_of(deg_ref)
    s = acc_ref[0] + acc_ref[1] + g_ref[...]
    h = jnp.maximum(s * dinv[:, None] + b_ref[...], 0.0)
    b_blk = batch_ref[0, 0, :]
    row_id = lax.broadcasted_iota(jnp.int32, (_BLK, _G), 0) + i * _BLK
    onehot = (
        (b_blk[:, None] == lax.broadcasted_iota(jnp.int32, (_BLK, _G), 1))
        & (row_id < _N)
    ).astype(jnp.float32)
    psum = lax.dot_general(
        onehot, h, (((0,), (0,)), ((), ())), preferred_element_type=jnp.float32
    )
    pcnt = jnp.sum(onehot, axis=0)

    @pl.when(i == 0)
    def _():
        sums_ref[...] = psum
        cnt_ref[...] = pcnt

    @pl.when(i > 0)
    def _():
        sums_ref[...] += psum
        cnt_ref[...] += pcnt

    @pl.when(i == _GRID - 1)
    def _():
        mean = sums_ref[...] / jnp.maximum(cnt_ref[...], 1.0)[:, None]
        o_ref[...] = (
            jnp.dot(mean, lw_ref[...], preferred_element_type=jnp.float32)
            + lb_ref[...]
        )


def _deg_spec():
    return pl.BlockSpec((_NC, _BLK, 16), lambda i: (0, i, 0))


def _row_spec():
    return pl.BlockSpec((_BLK, _D), lambda i: (i, 0))


def _acc_spec():
    return pl.BlockSpec((_NC, _BLK, _D), lambda i: (0, i, 0))


def _full_spec(shape):
    nd = len(shape)
    return pl.BlockSpec(shape, lambda i: (0,) * nd)


def _tc_first(degp, x, W):
    return pl.pallas_call(
        _first_body,
        grid=(_GRID,),
        in_specs=[_deg_spec(), _row_spec(), _full_spec((_D, _D))],
        out_specs=_row_spec(),
        out_shape=jax.ShapeDtypeStruct((_NPAD, _D), jnp.float32),
    )(degp, x, W)


def _tc_mid(degp, accp, g, W, b):
    return pl.pallas_call(
        _mid_body,
        grid=(_GRID,),
        in_specs=[_deg_spec(), _acc_spec(), _row_spec(),
                  _full_spec((_D, _D)), _full_spec((1, _D))],
        out_specs=_row_spec(),
        out_shape=jax.ShapeDtypeStruct((_NPAD, _D), jnp.float32),
    )(degp, accp, g, W, b)


def _tc_pool(degp, accp, g, b, batch3d, lw, lb):
    return pl.pallas_call(
        _pool_body,
        grid=(_GRID,),
        in_specs=[_deg_spec(), _acc_spec(), _row_spec(), _full_spec((1, _D)),
                  pl.BlockSpec((1, 1, _BLK), lambda i: (i, 0, 0)),
                  _full_spec((_D, _D)), _full_spec((1, _D))],
        out_specs=_full_spec((_G, _D)),
        out_shape=jax.ShapeDtypeStruct((_G, _D), jnp.float32),
        scratch_shapes=[
            pltpu.VMEM((_G, _D), jnp.float32),
            pltpu.VMEM((_G,), jnp.float32),
        ],
    )(degp, accp, g, b, batch3d, lw, lb)


def kernel(x, edge_index, batch, W1, b1, W2, b2, lin_W, lin_b):
    src = edge_index[0].astype(jnp.int32)
    dst = edge_index[1].astype(jnp.int32)
    x = jnp.pad(x, ((0, _NPAD - _N), (0, 0)))
    batch3d = (
        jnp.pad(batch.astype(jnp.int32), (0, _NPAD - _N))
        .reshape(_GRID, 1, _BLK)
    )
    zeros_rows = jnp.zeros((_RPT, _D), jnp.float32)
    zeros16 = jnp.zeros((_RPT, 16), jnp.float32)
    b1r = b1.reshape(1, _D)
    b2r = b2.reshape(1, _D)
    lbr = lin_b.reshape(1, _D)

    degp = _deg_kernel(dst, zeros16)
    g1 = _tc_first(degp, x, W1)
    acc1 = _edge_scatter(g1, src, dst, zeros_rows)
    g2 = _tc_mid(degp, acc1, g1, W2, b1r)
    acc2 = _edge_scatter(g2, src, dst, zeros_rows)
    return _tc_pool(degp, acc2, g2, b2r, batch3d, lin_W, lbr)


# speedup vs baseline: 2.4634x; 1.2817x over previous
import functools

import jax
import jax.numpy as jnp
from jax import lax
from jax.experimental import pallas as pl
from jax.experimental.pallas import tpu as pltpu
from jax.experimental.pallas import tpu_sc as plsc

_N = 10000
_NPAD = 10240
_E = 320000
_D = 128
_G = 64
_NC = 2
_NS = 16
_NW = _NC * _NS
_EPW = _E // _NW
_CH = 128
_NFULL = _EPW // _CH
_REM = _EPW - _NFULL * _CH
_RPT = _NPAD // _NS


def _sc_mesh():
    return plsc.VectorSubcoreMesh(core_axis_name="c", subcore_axis_name="s")


def _deg_kernel(dst, zeros16):
    @functools.partial(
        pl.kernel,
        mesh=_sc_mesh(),
        out_type=jax.ShapeDtypeStruct((_NC, _NPAD, 16), jnp.float32),
        compiler_params=pltpu.CompilerParams(use_tc_tiling_on_sc=False),
        scratch_types=[
            pltpu.VMEM((_CH,), jnp.int32),
            pltpu.VMEM((_REM,), jnp.int32),
            pltpu.VMEM((_CH, 16), jnp.float32),
            pltpu.VMEM_SHARED((_NPAD, 16), jnp.float32),
        ],
    )
    def k(dst_hbm, z_hbm, out_hbm, didx, didx2, ones_v, acc):
        cid = lax.axis_index("c")
        sid = lax.axis_index("s")
        wid = sid * _NC + cid
        base = wid * _EPW
        one_row = jnp.where(
            lax.iota(jnp.int32, 16) == 0, jnp.float32(1.0), jnp.float32(0.0)
        )

        def fill(i, carry):
            ones_v[i, :] = one_row
            return carry

        lax.fori_loop(0, _CH, fill, 0)
        pltpu.sync_copy(z_hbm, acc.at[pl.ds(sid * _RPT, _RPT)])
        plsc.subcore_barrier()

        def body(c, carry):
            off = base + c * _CH
            pltpu.sync_copy(dst_hbm.at[pl.ds(off, _CH)], didx)
            pltpu.sync_copy(ones_v, acc.at[didx], add=True)
            return carry

        lax.fori_loop(0, _NFULL, body, 0)
        off = base + _NFULL * _CH
        pltpu.sync_copy(dst_hbm.at[pl.ds(off, _REM)], didx2)
        pltpu.sync_copy(ones_v.at[pl.ds(0, _REM)], acc.at[didx2], add=True)
        plsc.subcore_barrier()
        pltpu.sync_copy(
            acc.at[pl.ds(sid * _RPT, _RPT)],
            out_hbm.at[cid, pl.ds(sid * _RPT, _RPT)],
        )

    return k(dst, zeros16)


def _edge_scatter(g, src, dst, zeros_rows):
    @functools.partial(
        pl.kernel,
        mesh=_sc_mesh(),
        out_type=jax.ShapeDtypeStruct((_NC, _NPAD, _D), jnp.float32),
        scratch_types=[
            pltpu.VMEM((_CH,), jnp.int32),
            pltpu.VMEM((_CH,), jnp.int32),
            pltpu.VMEM((_CH,), jnp.int32),
            pltpu.VMEM((_CH,), jnp.int32),
            pltpu.VMEM((_REM,), jnp.int32),
            pltpu.VMEM((_REM,), jnp.int32),
            pltpu.VMEM((_CH, _D), jnp.float32),
            pltpu.VMEM_SHARED((_NPAD, _D), jnp.float32),
            pltpu.SemaphoreType.DMA,
            pltpu.SemaphoreType.DMA,
            pltpu.SemaphoreType.DMA,
        ],
    )
    def k(g_hbm, src_hbm, dst_hbm, z_hbm, out_hbm,
          sidx0, sidx1, didx0, didx1, sidxr, didxr, rows, acc, gs, is0, is1):
        cid = lax.axis_index("c")
        sid = lax.axis_index("s")
        wid = sid * _NC + cid
        base = wid * _EPW
        pltpu.sync_copy(z_hbm, acc.at[pl.ds(sid * _RPT, _RPT)])
        plsc.subcore_barrier()

        sidx = (sidx0, sidx1)
        didx = (didx0, didx1)
        isem = (is0, is1)

        def idx_load(cc, p):
            off = base + cc * _CH
            pltpu.async_copy(src_hbm.at[pl.ds(off, _CH)], sidx[p], isem[p])
            pltpu.async_copy(dst_hbm.at[pl.ds(off, _CH)], didx[p], isem[p])

        def wait_idx(p):
            pltpu.make_async_copy(
                src_hbm.at[pl.ds(base, _CH)], sidx[p], isem[p]
            ).wait()
            pltpu.make_async_copy(
                dst_hbm.at[pl.ds(base, _CH)], didx[p], isem[p]
            ).wait()

        def chunk(cc, p, prefetch):
            wait_idx(p)
            pltpu.async_copy(g_hbm.at[sidx[p]], rows, gs).wait()
            if prefetch:
                idx_load(cc + 1, 1 - p)
            pltpu.sync_copy(rows, acc.at[didx[p]], add=True)

        idx_load(0, 0)

        def pair(i, carry):
            c = i * 2
            chunk(c, 0, True)
            chunk(c + 1, 1, True)
            return carry

        lax.fori_loop(0, (_NFULL - 2) // 2, pair, 0)
        chunk(_NFULL - 2, 0, True)
        chunk(_NFULL - 1, 1, False)

        off = base + _NFULL * _CH
        pltpu.sync_copy(src_hbm.at[pl.ds(off, _REM)], sidxr)
        pltpu.sync_copy(dst_hbm.at[pl.ds(off, _REM)], didxr)
        pltpu.async_copy(g_hbm.at[sidxr], rows.at[pl.ds(0, _REM)], gs).wait()
        pltpu.sync_copy(rows.at[pl.ds(0, _REM)], acc.at[didxr], add=True)
        plsc.subcore_barrier()
        pltpu.sync_copy(
            acc.at[pl.ds(sid * _RPT, _RPT)],
            out_hbm.at[cid, pl.ds(sid * _RPT, _RPT)],
        )

    return k(g, src, dst, zeros_rows)


_BLK = 2048
_GRID = _NPAD // _BLK


def _dinv_of(deg_ref):
    deg = jnp.sum(deg_ref[...], axis=(0, 2)) + 1.0
    return lax.rsqrt(deg)


def _first_body(deg_ref, x_ref, w_ref, o_ref):
    dinv = _dinv_of(deg_ref)
    h = jnp.dot(x_ref[...], w_ref[...], preferred_element_type=jnp.float32)
    o_ref[...] = h * dinv[:, None]


def _mid_body(deg_ref, acc_ref, g_ref, w_ref, b_ref, o_ref):
    dinv = _dinv_of(deg_ref)
    s = acc_ref[0] + acc_ref[1] + g_ref[...]
    h = jnp.maximum(s * dinv[:, None] + b_ref[...], 0.0)
    o_ref[...] = (
        jnp.dot(h, w_ref[...], preferred_element_type=jnp.float32) * dinv[:, None]
    )


def _pool_body(deg_ref, acc_ref, g_ref, b_ref, batch_ref, lw_ref, lb_ref,
               o_ref, sums_ref, cnt_ref):
    i = pl.program_id(0)
    dinv = _dinv_of(deg_ref)
    s = acc_ref[0] + acc_ref[1] + g_ref[...]
    h = jnp.maximum(s * dinv[:, None] + b_ref[...], 0.0)
    b_blk = batch_ref[0, 0, :]
    row_id = lax.broadcasted_iota(jnp.int32, (_BLK, _G), 0) + i * _BLK
    onehot = (
        (b_blk[:, None] == lax.broadcasted_iota(jnp.int32, (_BLK, _G), 1))
        & (row_id < _N)
    ).astype(jnp.float32)
    psum = lax.dot_general(
        onehot, h, (((0,), (0,)), ((), ())), preferred_element_type=jnp.float32
    )
    pcnt = jnp.sum(onehot, axis=0)

    @pl.when(i == 0)
    def _():
        sums_ref[...] = psum
        cnt_ref[...] = pcnt

    @pl.when(i > 0)
    def _():
        sums_ref[...] += psum
        cnt_ref[...] += pcnt

    @pl.when(i == _GRID - 1)
    def _():
        mean = sums_ref[...] / jnp.maximum(cnt_ref[...], 1.0)[:, None]
        o_ref[...] = (
            jnp.dot(mean, lw_ref[...], preferred_element_type=jnp.float32)
            + lb_ref[...]
        )


def _deg_spec():
    return pl.BlockSpec((_NC, _BLK, 16), lambda i: (0, i, 0))


def _row_spec():
    return pl.BlockSpec((_BLK, _D), lambda i: (i, 0))


def _acc_spec():
    return pl.BlockSpec((_NC, _BLK, _D), lambda i: (0, i, 0))


def _full_spec(shape):
    nd = len(shape)
    return pl.BlockSpec(shape, lambda i: (0,) * nd)


def _tc_first(degp, x, W):
    return pl.pallas_call(
        _first_body,
        grid=(_GRID,),
        in_specs=[_deg_spec(), _row_spec(), _full_spec((_D, _D))],
        out_specs=_row_spec(),
        out_shape=jax.ShapeDtypeStruct((_NPAD, _D), jnp.float32),
    )(degp, x, W)


def _tc_mid(degp, accp, g, W, b):
    return pl.pallas_call(
        _mid_body,
        grid=(_GRID,),
        in_specs=[_deg_spec(), _acc_spec(), _row_spec(),
                  _full_spec((_D, _D)), _full_spec((1, _D))],
        out_specs=_row_spec(),
        out_shape=jax.ShapeDtypeStruct((_NPAD, _D), jnp.float32),
    )(degp, accp, g, W, b)


def _tc_pool(degp, accp, g, b, batch3d, lw, lb):
    return pl.pallas_call(
        _pool_body,
        grid=(_GRID,),
        in_specs=[_deg_spec(), _acc_spec(), _row_spec(), _full_spec((1, _D)),
                  pl.BlockSpec((1, 1, _BLK), lambda i: (i, 0, 0)),
                  _full_spec((_D, _D)), _full_spec((1, _D))],
        out_specs=_full_spec((_G, _D)),
        out_shape=jax.ShapeDtypeStruct((_G, _D), jnp.float32),
        scratch_shapes=[
            pltpu.VMEM((_G, _D), jnp.float32),
            pltpu.VMEM((_G,), jnp.float32),
        ],
    )(degp, accp, g, b, batch3d, lw, lb)


def kernel(x, edge_index, batch, W1, b1, W2, b2, lin_W, lin_b):
    src = edge_index[0].astype(jnp.int32)
    dst = edge_index[1].astype(jnp.int32)
    x = jnp.pad(x, ((0, _NPAD - _N), (0, 0)))
    batch3d = (
        jnp.pad(batch.astype(jnp.int32), (0, _NPAD - _N))
        .reshape(_GRID, 1, _BLK)
    )
    zeros_rows = jnp.zeros((_RPT, _D), jnp.float32)
    zeros16 = jnp.zeros((_RPT, 16), jnp.float32)
    b1r = b1.reshape(1, _D)
    b2r = b2.reshape(1, _D)
    lbr = lin_b.reshape(1, _D)

    degp = _deg_kernel(dst, zeros16)
    g1 = _tc_first(degp, x, W1)
    acc1 = _edge_scatter(g1, src, dst, zeros_rows)
    g2 = _tc_mid(degp, acc1, g1, W2, b1r)
    acc2 = _edge_scatter(g2, src, dst, zeros_rows)
    return _tc_pool(degp, acc2, g2, b2r, batch3d, lin_W, lbr)
